# baseline (speedup 1.0000x reference)
import jax
import jax.numpy as jnp
from jax import lax
from jax.experimental import pallas as pl
from jax.experimental.pallas import tpu as pltpu
from jax.experimental.pallas import tpu_sc as plsc

_D = 256
_B_TOTAL = 32 * 1024
_NC, _NS = 2, 16
_NW = _NC * _NS
_B_PER_W = _B_TOTAL // _NW
_CHUNK = 64
_NCHUNK = _B_PER_W // _CHUNK
_NBUF = 4


def _gather_body(idx_hbm, table_hbm, out_hbm, idx_v, rows_v, gsems, wsems, wsems2):
    wid = lax.axis_index("s") * _NC + lax.axis_index("c")
    pltpu.sync_copy(idx_hbm.at[wid], idx_v)

    def start(i, b):
        pltpu.async_copy(
            table_hbm.at[idx_v.at[pl.ds(i * _CHUNK, _CHUNK)]],
            rows_v.at[b],
            gsems.at[b],
        )

    for b in range(_NBUF):
        start(b, b)

    def group(g, carry):
        i0 = g * _NBUF
        for b in range(_NBUF):
            i = i0 + b
            pltpu.make_async_copy(
                table_hbm.at[idx_v.at[pl.ds(i * _CHUNK, _CHUNK)]],
                rows_v.at[b],
                gsems.at[b],
            ).wait()
            h = _CHUNK // 2
            pltpu.async_copy(
                rows_v.at[b, pl.ds(0, h)],
                out_hbm.at[wid, pl.ds(i * _CHUNK, h)],
                wsems.at[b],
            )
            pltpu.async_copy(
                rows_v.at[b, pl.ds(h, h)],
                out_hbm.at[wid, pl.ds(i * _CHUNK + h, h)],
                wsems2.at[b],
            )

            @pl.when(i + _NBUF < _NCHUNK)
            def _():
                pltpu.make_async_copy(
                    rows_v.at[b, pl.ds(0, h)],
                    out_hbm.at[wid, pl.ds(i * _CHUNK, h)],
                    wsems.at[b],
                ).wait()
                pltpu.make_async_copy(
                    rows_v.at[b, pl.ds(h, h)],
                    out_hbm.at[wid, pl.ds(i * _CHUNK + h, h)],
                    wsems2.at[b],
                ).wait()
                start(i + _NBUF, b)

        return carry

    lax.fori_loop(0, _NCHUNK // _NBUF, group, 0)

    h = _CHUNK // 2
    for b in range(_NBUF):
        i = _NCHUNK - _NBUF + b
        pltpu.make_async_copy(
            rows_v.at[b, pl.ds(0, h)],
            out_hbm.at[wid, pl.ds(i * _CHUNK, h)],
            wsems.at[b],
        ).wait()
        pltpu.make_async_copy(
            rows_v.at[b, pl.ds(h, h)],
            out_hbm.at[wid, pl.ds(i * _CHUNK + h, h)],
            wsems2.at[b],
        ).wait()


@jax.jit
def _gather(x, table):
    mesh = plsc.VectorSubcoreMesh(core_axis_name="c", subcore_axis_name="s")
    return pl.kernel(
        _gather_body,
        mesh=mesh,
        out_type=jax.ShapeDtypeStruct((_NW, _B_PER_W, _D), jnp.float32),
        scratch_types=[
            pltpu.VMEM((_B_PER_W,), jnp.int32),
            pltpu.VMEM((_NBUF, _CHUNK, _D), jnp.float32),
            pltpu.SemaphoreType.DMA((_NBUF,)),
            pltpu.SemaphoreType.DMA((_NBUF,)),
            pltpu.SemaphoreType.DMA((_NBUF,)),
        ],
    )(x, table)


def kernel(x, table):
    return _gather(x.astype(jnp.int32), table)


# speedup vs baseline: 1.0138x; 1.0138x over previous
import jax
import jax.numpy as jnp
from jax import lax
from jax.experimental import pallas as pl
from jax.experimental.pallas import tpu as pltpu
from jax.experimental.pallas import tpu_sc as plsc

_D = 256
_B_TOTAL = 32 * 1024
_NC, _NS = 2, 16
_NW = _NC * _NS
_B_PER_W = _B_TOTAL // _NW
_CHUNK = 64
_NCHUNK = _B_PER_W // _CHUNK
_NBUF = 4


def _gather_body(idx_hbm, table_hbm, out_hbm, idx_v, rows_v, gsems, wsems):
    wid = lax.axis_index("s") * _NC + lax.axis_index("c")
    pltpu.sync_copy(idx_hbm.at[wid], idx_v)

    def start(i, b):
        pltpu.async_copy(
            table_hbm.at[idx_v.at[pl.ds(i * _CHUNK, _CHUNK)]],
            rows_v.at[b],
            gsems.at[b],
        )

    for b in range(_NBUF):
        start(b, b)

    def group(g, carry):
        i0 = g * _NBUF
        for b in range(_NBUF):
            i = i0 + b
            pltpu.make_async_copy(
                table_hbm.at[idx_v.at[pl.ds(i * _CHUNK, _CHUNK)]],
                rows_v.at[b],
                gsems.at[b],
            ).wait()
            pltpu.async_copy(
                rows_v.at[b],
                out_hbm.at[wid, pl.ds(i * _CHUNK, _CHUNK)],
                wsems.at[b],
            )

            @pl.when(i + _NBUF < _NCHUNK)
            def _():
                pltpu.make_async_copy(
                    rows_v.at[b],
                    out_hbm.at[wid, pl.ds(i * _CHUNK, _CHUNK)],
                    wsems.at[b],
                ).wait()
                start(i + _NBUF, b)

        return carry

    lax.fori_loop(0, _NCHUNK // _NBUF, group, 0)

    for b in range(_NBUF):
        pltpu.make_async_copy(
            rows_v.at[b],
            out_hbm.at[wid, pl.ds((_NCHUNK - _NBUF + b) * _CHUNK, _CHUNK)],
            wsems.at[b],
        ).wait()


@jax.jit
def _gather(x, table):
    mesh = plsc.VectorSubcoreMesh(core_axis_name="c", subcore_axis_name="s")
    return pl.kernel(
        _gather_body,
        mesh=mesh,
        out_type=jax.ShapeDtypeStruct((_NW, _B_PER_W, _D), jnp.float32),
        scratch_types=[
            pltpu.VMEM((_B_PER_W,), jnp.int32),
            pltpu.VMEM((_NBUF, _CHUNK, _D), jnp.float32),
            pltpu.SemaphoreType.DMA((_NBUF,)),
            pltpu.SemaphoreType.DMA((_NBUF,)),
        ],
    )(x, table)


def kernel(x, table):
    return _gather(x.astype(jnp.int32), table)
